# baseline (device time: 102881 ns/iter reference)
import os

import jax
import jax.numpy as jnp
from jax import lax
from jax.experimental import pallas as pl
from jax.experimental.pallas import tpu as pltpu

_COMM = os.environ.get("SCB_COMM", "1") == "1"

N_DEV = 32
ROWS = 512
D_MODEL = 1024
D_HEAD = 128
N_HEADS = 8
SKV = 2048
CHUNK = ROWS // N_DEV
NBLK = 4
BLK = ROWS // NBLK
CPB = N_DEV // NBLK
SCALE = 0.08838834764831843

_F32 = jnp.float32
_BF16 = jnp.bfloat16
_MESH = pltpu.DeviceIdType.MESH


def _body(x_ref, wq_ref, wo_ref, khbm_ref, vhbm_ref, out_ref,
          k_ref, v_ref, p_ref, pbf_ref, rs_buf, agbf_ref,
          kv_sems, rs_send, rs_recv, ag_send, ag_recv):
    my = lax.axis_index("i")

    kv_copies = []
    for g in range(2):
        for src, dst in ((khbm_ref, k_ref), (vhbm_ref, v_ref)):
            c = pltpu.make_async_copy(
                src.at[0, :, 2 * my + g, :], dst.at[g], kv_sems.at[len(kv_copies)]
            )
            c.start()
            kv_copies.append(c)

    for blk in range(NBLK):
        r0 = blk * BLK
        qb = jnp.dot(x_ref[0, r0:r0 + BLK, :], wq_ref[...],
                     preferred_element_type=_F32) * SCALE
        if blk == 0:
            for c in kv_copies:
                c.wait()
        heads = []
        for h in range(N_HEADS):
            g = h // 4
            qh = qb[:, D_HEAD * h:D_HEAD * (h + 1)]
            s = lax.dot_general(
                qh, k_ref[g], (((1,), (1,)), ((), ())),
                preferred_element_type=_F32,
            )
            e = jnp.exp(s)
            l = jnp.sum(e, axis=1, keepdims=True)
            o = jnp.dot(e, v_ref[g], preferred_element_type=_F32)
            heads.append(o / l)
        attn_blk = jnp.concatenate(heads, axis=1)
        pb = jnp.dot(attn_blk, wo_ref[...], preferred_element_type=_F32)
        p_ref[r0:r0 + BLK, :] = pb
        pbf_ref[r0:r0 + BLK, :] = pb.astype(_BF16)
        if not _COMM:
            out_ref[0, r0:r0 + BLK, :] = pb
            continue

        for q in range(1, N_DEV):
            d = lax.rem(my + q, N_DEV)

            @pl.when(lax.div(d, CPB) == blk)
            def _(d=d, q=q):
                pltpu.make_async_remote_copy(
                    src_ref=pbf_ref.at[pl.ds(d * CHUNK, CHUNK), :],
                    dst_ref=rs_buf.at[q - 1],
                    send_sem=rs_send.at[q - 1],
                    recv_sem=rs_recv.at[q - 1],
                    device_id=(d,),
                    device_id_type=_MESH,
                ).start()

        @pl.when(lax.div(my, CPB) == blk)
        def _(blk=blk):
            acc = p_ref[pl.ds(my * CHUNK, CHUNK), :]
            for j in range(N_DEV - 1):
                rv = pltpu.make_async_remote_copy(
                    src_ref=rs_buf.at[j], dst_ref=rs_buf.at[j],
                    send_sem=rs_send.at[j], recv_sem=rs_recv.at[j],
                    device_id=(my,), device_id_type=_MESH,
                )
                rv.wait_recv()
                acc = acc + rs_buf[j].astype(_F32)
            out_ref[0, pl.ds(my * CHUNK, CHUNK), :] = acc
            agbf_ref[pl.ds(my * CHUNK, CHUNK), :] = acc.astype(_BF16)
            for q in range(1, N_DEV):
                d = lax.rem(my + q, N_DEV)
                pltpu.make_async_remote_copy(
                    src_ref=agbf_ref.at[pl.ds(my * CHUNK, CHUNK), :],
                    dst_ref=agbf_ref.at[pl.ds(my * CHUNK, CHUNK), :],
                    send_sem=ag_send.at[q - 1],
                    recv_sem=ag_recv.at[q - 1],
                    device_id=(d,),
                    device_id_type=_MESH,
                ).start()

    if not _COMM:
        return

    for j in range(N_DEV - 1):
        s_dev = lax.rem(my + N_DEV - (j + 1), N_DEV)
        av = pltpu.make_async_remote_copy(
            src_ref=agbf_ref.at[pl.ds(s_dev * CHUNK, CHUNK), :],
            dst_ref=agbf_ref.at[pl.ds(s_dev * CHUNK, CHUNK), :],
            send_sem=ag_send.at[j], recv_sem=ag_recv.at[j],
            device_id=(my,), device_id_type=_MESH,
        )
        av.wait_recv()
        out_ref[0, pl.ds(s_dev * CHUNK, CHUNK), :] = (
            agbf_ref[pl.ds(s_dev * CHUNK, CHUNK), :].astype(_F32))

    for q in range(1, N_DEV):
        pltpu.make_async_remote_copy(
            src_ref=pbf_ref.at[pl.ds(0, CHUNK), :],
            dst_ref=rs_buf.at[q - 1],
            send_sem=rs_send.at[q - 1], recv_sem=rs_recv.at[q - 1],
            device_id=(my,), device_id_type=_MESH,
        ).wait_send()
        pltpu.make_async_remote_copy(
            src_ref=agbf_ref.at[pl.ds(0, CHUNK), :],
            dst_ref=agbf_ref.at[pl.ds(0, CHUNK), :],
            send_sem=ag_send.at[q - 1], recv_sem=ag_recv.at[q - 1],
            device_id=(my,), device_id_type=_MESH,
        ).wait_send()


def kernel(x, Wq, Wo, K_ext, V_ext):
    return pl.pallas_call(
        _body,
        out_shape=jax.ShapeDtypeStruct((1, ROWS, D_MODEL), _F32),
        in_specs=[
            pl.BlockSpec(memory_space=pltpu.VMEM),
            pl.BlockSpec(memory_space=pltpu.VMEM),
            pl.BlockSpec(memory_space=pltpu.VMEM),
            pl.BlockSpec(memory_space=pltpu.MemorySpace.HBM),
            pl.BlockSpec(memory_space=pltpu.MemorySpace.HBM),
        ],
        out_specs=pl.BlockSpec(memory_space=pltpu.VMEM),
        scratch_shapes=[
            pltpu.VMEM((2, SKV, D_HEAD), _F32),
            pltpu.VMEM((2, SKV, D_HEAD), _F32),
            pltpu.VMEM((ROWS, D_MODEL), _F32),
            pltpu.VMEM((ROWS, D_MODEL), _BF16),
            pltpu.VMEM((N_DEV - 1, CHUNK, D_MODEL), _BF16),
            pltpu.VMEM((ROWS, D_MODEL), _BF16),
            pltpu.SemaphoreType.DMA((4,)),
            pltpu.SemaphoreType.DMA((N_DEV - 1,)),
            pltpu.SemaphoreType.DMA((N_DEV - 1,)),
            pltpu.SemaphoreType.DMA((N_DEV - 1,)),
            pltpu.SemaphoreType.DMA((N_DEV - 1,)),
        ],
        compiler_params=pltpu.CompilerParams(
            vmem_limit_bytes=100 * 1024 * 1024,
        ),
    )(x, Wq, Wo, K_ext, V_ext)


# device time: 66729 ns/iter; 1.5418x vs baseline; 1.5418x over previous
import os

import jax
import jax.numpy as jnp
from jax import lax
from jax.experimental import pallas as pl
from jax.experimental.pallas import tpu as pltpu

_COMM = os.environ.get("SCB_COMM", "1") == "1"

N_DEV = 32
ROWS = 512
D_MODEL = 1024
D_HEAD = 128
N_HEADS = 8
SKV = 2048
CHUNK = ROWS // N_DEV
NBLK = 4
BLK = ROWS // NBLK
CPB = N_DEV // NBLK
SCALE = 0.08838834764831843

_F32 = jnp.float32
_BF16 = jnp.bfloat16
_MESH = pltpu.DeviceIdType.MESH


def _body(x_ref, wq_ref, wo_ref, khbm_ref, vhbm_ref, out_ref,
          k_ref, vaug_ref, p_ref, pbf_ref, rs_buf, agbf_ref,
          kv_sems, rs_send, rs_recv, ag_send, ag_recv):
    my = lax.axis_index("i")

    kv_copies = []
    for g in range(2):
        for src, dst in (
            (khbm_ref, k_ref.at[g]),
            (vhbm_ref, vaug_ref.at[g, :, pl.ds(0, D_HEAD)]),
        ):
            c = pltpu.make_async_copy(
                src.at[0, :, 2 * my + g, :], dst, kv_sems.at[len(kv_copies)]
            )
            c.start()
            kv_copies.append(c)
        vaug_ref[g, :, D_HEAD:] = jnp.where(
            lax.broadcasted_iota(jnp.int32, (SKV, D_HEAD), 1) == 0, 1.0, 0.0
        )

    for blk in range(NBLK):
        r0 = blk * BLK
        qb = jnp.dot(x_ref[0, r0:r0 + BLK, :], wq_ref[...],
                     preferred_element_type=_F32) * SCALE
        if blk == 0:
            for c in kv_copies:
                c.wait()
        heads = []
        for h in range(N_HEADS):
            g = h // 4
            qh = qb[:, D_HEAD * h:D_HEAD * (h + 1)]
            s = lax.dot_general(
                qh, k_ref[g], (((1,), (1,)), ((), ())),
                preferred_element_type=_F32,
            )
            e = jnp.exp(s)
            oa = jnp.dot(e, vaug_ref[g], preferred_element_type=_F32)
            heads.append(oa[:, :D_HEAD] / oa[:, D_HEAD:D_HEAD + 1])
        attn_blk = jnp.concatenate(heads, axis=1)
        pb = jnp.dot(attn_blk, wo_ref[...], preferred_element_type=_F32)
        p_ref[r0:r0 + BLK, :] = pb
        pbf_ref[r0:r0 + BLK, :] = pb.astype(_BF16)
        if not _COMM:
            out_ref[0, r0:r0 + BLK, :] = pb
            continue

        for q in range(1, N_DEV):
            d = lax.rem(my + q, N_DEV)

            @pl.when(lax.div(d, CPB) == blk)
            def _(d=d, q=q):
                pltpu.make_async_remote_copy(
                    src_ref=pbf_ref.at[pl.ds(d * CHUNK, CHUNK), :],
                    dst_ref=rs_buf.at[q - 1],
                    send_sem=rs_send.at[q - 1],
                    recv_sem=rs_recv.at[q - 1],
                    device_id=(d,),
                    device_id_type=_MESH,
                ).start()

    if not _COMM:
        return

    acc = p_ref[pl.ds(my * CHUNK, CHUNK), :]
    for j in range(N_DEV - 1):
        rv = pltpu.make_async_remote_copy(
            src_ref=rs_buf.at[j], dst_ref=rs_buf.at[j],
            send_sem=rs_send.at[j], recv_sem=rs_recv.at[j],
            device_id=(my,), device_id_type=_MESH,
        )
        rv.wait_recv()
        acc = acc + rs_buf[j].astype(_F32)
    agbf_ref[pl.ds(my * CHUNK, CHUNK), :] = acc.astype(_BF16)

    for q in range(1, N_DEV):
        d = lax.rem(my + q, N_DEV)
        pltpu.make_async_remote_copy(
            src_ref=agbf_ref.at[pl.ds(my * CHUNK, CHUNK), :],
            dst_ref=agbf_ref.at[pl.ds(my * CHUNK, CHUNK), :],
            send_sem=ag_send.at[q - 1],
            recv_sem=ag_recv.at[q - 1],
            device_id=(d,),
            device_id_type=_MESH,
        ).start()

    for j in range(N_DEV - 1):
        s_dev = lax.rem(my + N_DEV - (j + 1), N_DEV)
        av = pltpu.make_async_remote_copy(
            src_ref=agbf_ref.at[pl.ds(s_dev * CHUNK, CHUNK), :],
            dst_ref=agbf_ref.at[pl.ds(s_dev * CHUNK, CHUNK), :],
            send_sem=ag_send.at[j], recv_sem=ag_recv.at[j],
            device_id=(my,), device_id_type=_MESH,
        )
        av.wait_recv()
    out_ref[0] = agbf_ref[...].astype(_F32)
    out_ref[0, pl.ds(my * CHUNK, CHUNK), :] = acc

    for q in range(1, N_DEV):
        pltpu.make_async_remote_copy(
            src_ref=pbf_ref.at[pl.ds(0, CHUNK), :],
            dst_ref=rs_buf.at[q - 1],
            send_sem=rs_send.at[q - 1], recv_sem=rs_recv.at[q - 1],
            device_id=(my,), device_id_type=_MESH,
        ).wait_send()
        pltpu.make_async_remote_copy(
            src_ref=agbf_ref.at[pl.ds(0, CHUNK), :],
            dst_ref=agbf_ref.at[pl.ds(0, CHUNK), :],
            send_sem=ag_send.at[q - 1], recv_sem=ag_recv.at[q - 1],
            device_id=(my,), device_id_type=_MESH,
        ).wait_send()


def kernel(x, Wq, Wo, K_ext, V_ext):
    return pl.pallas_call(
        _body,
        out_shape=jax.ShapeDtypeStruct((1, ROWS, D_MODEL), _F32),
        in_specs=[
            pl.BlockSpec(memory_space=pltpu.VMEM),
            pl.BlockSpec(memory_space=pltpu.VMEM),
            pl.BlockSpec(memory_space=pltpu.VMEM),
            pl.BlockSpec(memory_space=pltpu.MemorySpace.HBM),
            pl.BlockSpec(memory_space=pltpu.MemorySpace.HBM),
        ],
        out_specs=pl.BlockSpec(memory_space=pltpu.VMEM),
        scratch_shapes=[
            pltpu.VMEM((2, SKV, D_HEAD), _F32),
            pltpu.VMEM((2, SKV, 2 * D_HEAD), _F32),
            pltpu.VMEM((ROWS, D_MODEL), _F32),
            pltpu.VMEM((ROWS, D_MODEL), _BF16),
            pltpu.VMEM((N_DEV - 1, CHUNK, D_MODEL), _BF16),
            pltpu.VMEM((ROWS, D_MODEL), _BF16),
            pltpu.SemaphoreType.DMA((4,)),
            pltpu.SemaphoreType.DMA((N_DEV - 1,)),
            pltpu.SemaphoreType.DMA((N_DEV - 1,)),
            pltpu.SemaphoreType.DMA((N_DEV - 1,)),
            pltpu.SemaphoreType.DMA((N_DEV - 1,)),
        ],
        compiler_params=pltpu.CompilerParams(
            vmem_limit_bytes=100 * 1024 * 1024,
        ),
    )(x, Wq, Wo, K_ext, V_ext)


# device time: 65281 ns/iter; 1.5760x vs baseline; 1.0222x over previous
import os

import jax
import jax.numpy as jnp
from jax import lax
from jax.experimental import pallas as pl
from jax.experimental.pallas import tpu as pltpu

_COMM = os.environ.get("SCB_COMM", "1") == "1"

N_DEV = 32
ROWS = 512
D_MODEL = 1024
D_HEAD = 128
N_HEADS = 8
SKV = 2048
CHUNK = ROWS // N_DEV
NBLK = 4
BLK = ROWS // NBLK
CPB = N_DEV // NBLK
SCALE = 0.08838834764831843

_F32 = jnp.float32
_BF16 = jnp.bfloat16
_MESH = pltpu.DeviceIdType.MESH


def _body(x_ref, wq_ref, wo_ref, khbm_ref, vhbm_ref, out_ref,
          k_ref, v_ref, p_ref, pbf_ref, rs_buf, agbf_ref,
          kv_sems, rs_send, rs_recv, ag_send, ag_recv):
    my = lax.axis_index("i")

    kv_copies = []
    for g in range(2):
        for src, dst in ((khbm_ref, k_ref.at[g]), (vhbm_ref, v_ref.at[g])):
            c = pltpu.make_async_copy(
                src.at[0, :, 2 * my + g, :], dst, kv_sems.at[len(kv_copies)]
            )
            c.start()
            kv_copies.append(c)

    for blk in range(NBLK):
        r0 = blk * BLK
        qb = jnp.dot(x_ref[0, r0:r0 + BLK, :], wq_ref[...],
                     preferred_element_type=_F32) * SCALE
        if blk == 0:
            for c in kv_copies:
                c.wait()
        heads = []
        for h in range(N_HEADS):
            g = h // 4
            qh = qb[:, D_HEAD * h:D_HEAD * (h + 1)]
            s = lax.dot_general(
                qh, k_ref[g], (((1,), (1,)), ((), ())),
                preferred_element_type=_F32,
            )
            e = jnp.exp(s)
            l = jnp.sum(e, axis=1, keepdims=True)
            o = jnp.dot(e, v_ref[g], preferred_element_type=_F32)
            heads.append(o / l)
        attn_blk = jnp.concatenate(heads, axis=1)
        pb = jnp.dot(attn_blk, wo_ref[...], preferred_element_type=_F32)
        p_ref[r0:r0 + BLK, :] = pb
        pbf_ref[r0:r0 + BLK, :] = pb.astype(_BF16)
        if not _COMM:
            out_ref[0, r0:r0 + BLK, :] = pb
            continue

        for q in range(1, N_DEV):
            d = lax.rem(my + q, N_DEV)

            @pl.when(lax.div(d, CPB) == blk)
            def _(d=d, q=q):
                pltpu.make_async_remote_copy(
                    src_ref=pbf_ref.at[pl.ds(d * CHUNK, CHUNK), :],
                    dst_ref=rs_buf.at[q - 1],
                    send_sem=rs_send.at[q - 1],
                    recv_sem=rs_recv.at[q - 1],
                    device_id=(d,),
                    device_id_type=_MESH,
                ).start()

    if not _COMM:
        return

    acc = p_ref[pl.ds(my * CHUNK, CHUNK), :]
    for j in range(N_DEV - 1):
        rv = pltpu.make_async_remote_copy(
            src_ref=rs_buf.at[j], dst_ref=rs_buf.at[j],
            send_sem=rs_send.at[j], recv_sem=rs_recv.at[j],
            device_id=(my,), device_id_type=_MESH,
        )
        rv.wait_recv()
        acc = acc + rs_buf[j].astype(_F32)
    agbf_ref[pl.ds(my * CHUNK, CHUNK), :] = acc.astype(_BF16)

    for q in range(1, N_DEV):
        d = lax.rem(my + q, N_DEV)
        pltpu.make_async_remote_copy(
            src_ref=agbf_ref.at[pl.ds(my * CHUNK, CHUNK), :],
            dst_ref=agbf_ref.at[pl.ds(my * CHUNK, CHUNK), :],
            send_sem=ag_send.at[q - 1],
            recv_sem=ag_recv.at[q - 1],
            device_id=(d,),
            device_id_type=_MESH,
        ).start()

    for j in range(N_DEV - 1):
        s_dev = lax.rem(my + N_DEV - (j + 1), N_DEV)
        av = pltpu.make_async_remote_copy(
            src_ref=agbf_ref.at[pl.ds(s_dev * CHUNK, CHUNK), :],
            dst_ref=agbf_ref.at[pl.ds(s_dev * CHUNK, CHUNK), :],
            send_sem=ag_send.at[j], recv_sem=ag_recv.at[j],
            device_id=(my,), device_id_type=_MESH,
        )
        av.wait_recv()
    out_ref[0] = agbf_ref[...].astype(_F32)
    out_ref[0, pl.ds(my * CHUNK, CHUNK), :] = acc

    for q in range(1, N_DEV):
        pltpu.make_async_remote_copy(
            src_ref=pbf_ref.at[pl.ds(0, CHUNK), :],
            dst_ref=rs_buf.at[q - 1],
            send_sem=rs_send.at[q - 1], recv_sem=rs_recv.at[q - 1],
            device_id=(my,), device_id_type=_MESH,
        ).wait_send()
        pltpu.make_async_remote_copy(
            src_ref=agbf_ref.at[pl.ds(0, CHUNK), :],
            dst_ref=agbf_ref.at[pl.ds(0, CHUNK), :],
            send_sem=ag_send.at[q - 1], recv_sem=ag_recv.at[q - 1],
            device_id=(my,), device_id_type=_MESH,
        ).wait_send()


def kernel(x, Wq, Wo, K_ext, V_ext):
    return pl.pallas_call(
        _body,
        out_shape=jax.ShapeDtypeStruct((1, ROWS, D_MODEL), _F32),
        in_specs=[
            pl.BlockSpec(memory_space=pltpu.VMEM),
            pl.BlockSpec(memory_space=pltpu.VMEM),
            pl.BlockSpec(memory_space=pltpu.VMEM),
            pl.BlockSpec(memory_space=pltpu.MemorySpace.HBM),
            pl.BlockSpec(memory_space=pltpu.MemorySpace.HBM),
        ],
        out_specs=pl.BlockSpec(memory_space=pltpu.VMEM),
        scratch_shapes=[
            pltpu.VMEM((2, SKV, D_HEAD), _F32),
            pltpu.VMEM((2, SKV, D_HEAD), _F32),
            pltpu.VMEM((ROWS, D_MODEL), _F32),
            pltpu.VMEM((ROWS, D_MODEL), _BF16),
            pltpu.VMEM((N_DEV - 1, CHUNK, D_MODEL), _BF16),
            pltpu.VMEM((ROWS, D_MODEL), _BF16),
            pltpu.SemaphoreType.DMA((4,)),
            pltpu.SemaphoreType.DMA((N_DEV - 1,)),
            pltpu.SemaphoreType.DMA((N_DEV - 1,)),
            pltpu.SemaphoreType.DMA((N_DEV - 1,)),
            pltpu.SemaphoreType.DMA((N_DEV - 1,)),
        ],
        compiler_params=pltpu.CompilerParams(
            vmem_limit_bytes=100 * 1024 * 1024,
        ),
    )(x, Wq, Wo, K_ext, V_ext)
